# Initial kernel scaffold; baseline (speedup 1.0000x reference)
import jax
import jax.numpy as jnp
from jax.experimental import pallas as pl


def kernel(x_list, edge_list, W_proj, b_proj, W_score, b_score):
    raise NotImplementedError("write your pallas kernel here")


# speedup vs baseline: 1.2862x; 1.2862x over previous
import jax
import jax.numpy as jnp
from jax import lax
from jax.experimental import pallas as pl


def _attn_body(x_ref, wscore_ref, wproj_ref, bproj_ref, out_ref):
    xb = x_ref[0]
    s_row = lax.dot_general(wscore_ref[...], xb,
                            (((1,), (1,)), ((), ())),
                            preferred_element_type=jnp.float32)
    m = jnp.max(s_row)
    e_row = jnp.exp(s_row - m)
    z = jnp.sum(e_row)
    xw = jnp.dot(e_row, xb, preferred_element_type=jnp.float32)
    xw = xw / z
    out = lax.dot_general(xw, wproj_ref[...],
                          (((1,), (1,)), ((), ())),
                          preferred_element_type=jnp.float32) + bproj_ref[...]
    out_ref[...] = out[None]


def kernel(x_list, edge_list, W_proj, b_proj, W_score, b_score):
    n, m, c = x_list.shape
    o = W_proj.shape[0]
    b_proj2 = b_proj.reshape(1, o)
    out = pl.pallas_call(
        _attn_body,
        grid=(n,),
        in_specs=[
            pl.BlockSpec((1, m, c), lambda i: (i, 0, 0)),
            pl.BlockSpec((1, c), lambda i: (0, 0)),
            pl.BlockSpec((o, c), lambda i: (0, 0)),
            pl.BlockSpec((1, o), lambda i: (0, 0)),
        ],
        out_specs=pl.BlockSpec((1, 1, o), lambda i: (i, 0, 0)),
        out_shape=jax.ShapeDtypeStruct((n, 1, o), jnp.float32),
    )(x_list, W_score, W_proj, b_proj2)
    return out.reshape(n, o)
